# baseline (speedup 1.0000x reference)
import functools

import jax
import jax.numpy as jnp
from jax.experimental import pallas as pl
from jax.experimental.pallas import tpu as pltpu

_B, _N, _FD = 4, 256, 128
_THRESHOLD = 0.5
_TI = 8
_BB = 2
_NJ = 128


def _body(H_ref, A_ref, W1a_ref, W1b_ref, b1_ref, w2_ref, b2_ref,
          rW1h_ref, rW1n_ref, rw1c_ref, rb1_ref, rW2_ref, rb2_ref,
          Hout_ref, mask_ref, XiS, PiS, EiS, zS):
    f32 = jnp.float32

    w2 = w2_ref[...]
    w2sum = jnp.sum(w2)
    NXj, Pj, Ej = [], [], []
    for bb in range(_BB):
        H = H_ref[bb]
        Xi = jnp.dot(H, W1a_ref[...], preferred_element_type=f32)
        Xj = jnp.dot(H, W1b_ref[...], preferred_element_type=f32) + b1_ref[...]
        XiS[bb] = Xi
        PiS[bb] = Xi * w2
        EiS[bb] = jnp.exp(Xi) * w2
        NXj.append(-Xj)
        Pj.append(Xj * w2 + w2)
        Ej.append(jnp.exp(Xj))

    for jh in range(_N // _NJ):
        jsl = slice(jh * _NJ, (jh + 1) * _NJ)
        NXjh = [a[jsl, :][None, :, :] for a in NXj]
        Pjh = [a[jsl, :][None, :, :] for a in Pj]
        Ejh = [a[jsl, :][None, :, :] for a in Ej]

        def tile_fn(t, _):
            sl = pl.ds(t * _TI, _TI)
            for bb in range(_BB):
                xi = XiS[bb, sl, :][:, None, :]
                pos = PiS[bb, sl, :][:, None, :] + Pjh[bb]
                neg = EiS[bb, sl, :][:, None, :] * Ejh[bb]
                e = jnp.where(xi > NXjh[bb], pos, neg)
                zS[bb, sl, jsl] = jnp.sum(e, axis=-1)
            return 0

        jax.lax.fori_loop(0, _N // _TI, tile_fn, 0, unroll=2)

    for bb in range(_BB):
        H = H_ref[bb]
        A = A_ref[bb]
        z = zS[bb]
        mask = A > 0.1
        zmax = jnp.max(jnp.where(mask, z, -1e30), axis=-1, keepdims=True)
        anym = jnp.max(A, axis=-1, keepdims=True) > 0.1
        b2 = b2_ref[0, 0]
        contra_max = jnp.where(anym, jax.nn.sigmoid(zmax - w2sum + b2), 0.0)
        cmask = contra_max > _THRESHOLD

        nb = (A > 0).astype(f32)
        cnt = jnp.sum(nb, axis=-1, keepdims=True)
        nbH = jnp.dot(nb, H, preferred_element_type=f32)
        neigh = nbH / jnp.maximum(cnt, 1.0)

        pre = (jnp.dot(H, rW1h_ref[...], preferred_element_type=f32)
               + jnp.dot(neigh, rW1n_ref[...], preferred_element_type=f32)
               + contra_max * rw1c_ref[...]
               + rb1_ref[...])
        act = jnp.where(pre > 0, pre, jnp.exp(pre) - 1.0)
        resolved = jnp.dot(act, rW2_ref[...], preferred_element_type=f32) + rb2_ref[...]

        apply = jnp.logical_and(cmask, cnt > 0)
        Hout_ref[bb] = jnp.where(apply, resolved, H)
        mask_ref[bb] = cmask.astype(jnp.int32)


@functools.partial(jax.jit, static_argnums=())
def kernel(H, A, det_W1, det_b1, det_W2, det_b2, res_W1, res_b1, res_W2, res_b2):
    B, N, FD = H.shape
    W1a = det_W1[:FD]
    W1b = det_W1[FD:]
    b1 = det_b1.reshape(1, FD)
    w2 = det_W2.reshape(1, FD)
    b2 = det_b2.reshape(1, 1)
    rW1h = res_W1[:FD]
    rW1n = res_W1[FD:2 * FD]
    rw1c = res_W1[2 * FD:2 * FD + 1]
    rb1 = res_b1.reshape(1, FD)
    rW2 = res_W2
    rb2 = res_b2.reshape(1, FD)

    full = lambda shape: pl.BlockSpec(shape, lambda b: (0,) * len(shape))
    Hout, mask32 = pl.pallas_call(
        _body,
        grid=(B // _BB,),
        in_specs=[
            pl.BlockSpec((_BB, N, FD), lambda b: (b, 0, 0)),
            pl.BlockSpec((_BB, N, N), lambda b: (b, 0, 0)),
            full((FD, FD)), full((FD, FD)), full((1, FD)), full((1, FD)),
            full((1, 1)),
            full((FD, FD)), full((FD, FD)), full((1, FD)), full((1, FD)),
            full((FD, FD)), full((1, FD)),
        ],
        out_specs=[
            pl.BlockSpec((_BB, N, FD), lambda b: (b, 0, 0)),
            pl.BlockSpec((_BB, N, 1), lambda b: (b, 0, 0)),
        ],
        out_shape=[
            jax.ShapeDtypeStruct((B, N, FD), jnp.float32),
            jax.ShapeDtypeStruct((B, N, 1), jnp.int32),
        ],
        compiler_params=pltpu.CompilerParams(
            dimension_semantics=("arbitrary",),
        ),
        scratch_shapes=[
            pltpu.VMEM((_BB, N, FD), jnp.float32),
            pltpu.VMEM((_BB, N, FD), jnp.float32),
            pltpu.VMEM((_BB, N, FD), jnp.float32),
            pltpu.VMEM((_BB, N, N), jnp.float32),
        ],
    )(H, A, W1a, W1b, b1, w2, b2, rW1h, rW1n, rw1c, rb1, rW2, rb2)
    return (Hout, mask32.reshape(B, N).astype(jnp.bool_))


# speedup vs baseline: 1.0978x; 1.0978x over previous
import functools

import jax
import jax.numpy as jnp
from jax.experimental import pallas as pl
from jax.experimental.pallas import tpu as pltpu

_B, _N, _FD = 4, 256, 128
_THRESHOLD = 0.5
_TI = 8
_BB = 4
_NJ = 128


def _body(H_ref, A_ref, W1a_ref, W1b_ref, b1_ref, w2_ref, b2_ref,
          rW1h_ref, rW1n_ref, rw1c_ref, rb1_ref, rW2_ref, rb2_ref,
          Hout_ref, mask_ref, XiS, PiS, EiS, zS):
    f32 = jnp.float32

    w2 = w2_ref[...]
    w2sum = jnp.sum(w2)
    NXj, Pj, Ej = [], [], []
    for bb in range(_BB):
        H = H_ref[bb]
        Xi = jnp.dot(H, W1a_ref[...], preferred_element_type=f32)
        Xj = jnp.dot(H, W1b_ref[...], preferred_element_type=f32) + b1_ref[...]
        XiS[bb] = Xi
        PiS[bb] = Xi * w2
        EiS[bb] = jnp.exp(Xi) * w2
        NXj.append(-Xj)
        Pj.append(Xj * w2 + w2)
        Ej.append(jnp.exp(Xj))

    NXj = [a[None, :, :] for a in NXj]
    Pj = [a[None, :, :] for a in Pj]
    Ej = [a[None, :, :] for a in Ej]

    def tile_fn(t, _):
        sl = pl.ds(t * _TI, _TI)
        for bb in range(_BB):
            xi = XiS[bb, sl, :][:, None, :]
            pos = PiS[bb, sl, :][:, None, :] + Pj[bb]
            neg = EiS[bb, sl, :][:, None, :] * Ej[bb]
            e = jnp.where(xi > NXj[bb], pos, neg)
            zS[bb, sl, :] = jnp.sum(e, axis=-1)
        return 0

    jax.lax.fori_loop(0, _N // _TI, tile_fn, 0, unroll=2)

    for bb in range(_BB):
        H = H_ref[bb]
        A = A_ref[bb]
        z = zS[bb]
        mask = A > 0.1
        zmax = jnp.max(jnp.where(mask, z, -1e30), axis=-1, keepdims=True)
        anym = jnp.max(A, axis=-1, keepdims=True) > 0.1
        b2 = b2_ref[0, 0]
        contra_max = jnp.where(anym, jax.nn.sigmoid(zmax - w2sum + b2), 0.0)
        cmask = contra_max > _THRESHOLD

        nb = (A > 0).astype(f32)
        cnt = jnp.sum(nb, axis=-1, keepdims=True)
        nbH = jnp.dot(nb, H, preferred_element_type=f32)
        neigh = nbH / jnp.maximum(cnt, 1.0)

        pre = (jnp.dot(H, rW1h_ref[...], preferred_element_type=f32)
               + jnp.dot(neigh, rW1n_ref[...], preferred_element_type=f32)
               + contra_max * rw1c_ref[...]
               + rb1_ref[...])
        act = jnp.where(pre > 0, pre, jnp.exp(pre) - 1.0)
        resolved = jnp.dot(act, rW2_ref[...], preferred_element_type=f32) + rb2_ref[...]

        apply = jnp.logical_and(cmask, cnt > 0)
        Hout_ref[bb] = jnp.where(apply, resolved, H)
        mask_ref[bb] = cmask.astype(jnp.int32)


@functools.partial(jax.jit, static_argnums=())
def kernel(H, A, det_W1, det_b1, det_W2, det_b2, res_W1, res_b1, res_W2, res_b2):
    B, N, FD = H.shape
    W1a = det_W1[:FD]
    W1b = det_W1[FD:]
    b1 = det_b1.reshape(1, FD)
    w2 = det_W2.reshape(1, FD)
    b2 = det_b2.reshape(1, 1)
    rW1h = res_W1[:FD]
    rW1n = res_W1[FD:2 * FD]
    rw1c = res_W1[2 * FD:2 * FD + 1]
    rb1 = res_b1.reshape(1, FD)
    rW2 = res_W2
    rb2 = res_b2.reshape(1, FD)

    full = lambda shape: pl.BlockSpec(shape, lambda b: (0,) * len(shape))
    Hout, mask32 = pl.pallas_call(
        _body,
        grid=(B // _BB,),
        in_specs=[
            pl.BlockSpec((_BB, N, FD), lambda b: (b, 0, 0)),
            pl.BlockSpec((_BB, N, N), lambda b: (b, 0, 0)),
            full((FD, FD)), full((FD, FD)), full((1, FD)), full((1, FD)),
            full((1, 1)),
            full((FD, FD)), full((FD, FD)), full((1, FD)), full((1, FD)),
            full((FD, FD)), full((1, FD)),
        ],
        out_specs=[
            pl.BlockSpec((_BB, N, FD), lambda b: (b, 0, 0)),
            pl.BlockSpec((_BB, N, 1), lambda b: (b, 0, 0)),
        ],
        out_shape=[
            jax.ShapeDtypeStruct((B, N, FD), jnp.float32),
            jax.ShapeDtypeStruct((B, N, 1), jnp.int32),
        ],
        compiler_params=pltpu.CompilerParams(
            dimension_semantics=("arbitrary",),
        ),
        scratch_shapes=[
            pltpu.VMEM((_BB, N, FD), jnp.float32),
            pltpu.VMEM((_BB, N, FD), jnp.float32),
            pltpu.VMEM((_BB, N, FD), jnp.float32),
            pltpu.VMEM((_BB, N, N), jnp.float32),
        ],
    )(H, A, W1a, W1b, b1, w2, b2, rW1h, rW1n, rw1c, rb1, rW2, rb2)
    return (Hout, mask32.reshape(B, N).astype(jnp.bool_))


# speedup vs baseline: 1.1203x; 1.0205x over previous
import functools

import jax
import jax.numpy as jnp
from jax.experimental import pallas as pl
from jax.experimental.pallas import tpu as pltpu

_B, _N, _FD = 4, 256, 128
_THRESHOLD = 0.5
_TI = 8
_BB = 4
_NJ = 128


def _body(H_ref, A_ref, W1a_ref, W1b_ref, b1_ref, w2_ref, b2_ref,
          rW1h_ref, rW1n_ref, rw1c_ref, rb1_ref, rW2_ref, rb2_ref,
          Hout_ref, mask_ref, XiS, PiS, EiS, zS):
    f32 = jnp.float32

    w2 = w2_ref[...]
    w2sum = jnp.sum(w2)
    NXj, Pj, Ej = [], [], []
    for bb in range(_BB):
        H = H_ref[bb]
        Xi = jnp.dot(H, W1a_ref[...], preferred_element_type=f32)
        Xj = jnp.dot(H, W1b_ref[...], preferred_element_type=f32) + b1_ref[...]
        XiS[bb] = Xi
        PiS[bb] = Xi * w2
        EiS[bb] = jnp.exp(Xi) * w2
        NXj.append(-Xj)
        Pj.append(Xj * w2 + w2)
        Ej.append(jnp.exp(Xj))

    NXj = [a[None, :, :] for a in NXj]
    Pj = [a[None, :, :] for a in Pj]
    Ej = [a[None, :, :] for a in Ej]

    def tile_fn(t, _):
        sl = pl.ds(t * _TI, _TI)
        for bb in range(_BB):
            xi = XiS[bb, sl, :][:, None, :]
            pos = PiS[bb, sl, :][:, None, :] + Pj[bb]
            neg = EiS[bb, sl, :][:, None, :] * Ej[bb]
            e = jnp.where(xi > NXj[bb], pos, neg)
            zS[bb, sl, :] = jnp.sum(e, axis=-1)
        return 0

    jax.lax.fori_loop(0, _N // _TI, tile_fn, 0, unroll=4)

    for bb in range(_BB):
        H = H_ref[bb]
        A = A_ref[bb]
        z = zS[bb]
        mask = A > 0.1
        zmax = jnp.max(jnp.where(mask, z, -1e30), axis=-1, keepdims=True)
        anym = jnp.max(A, axis=-1, keepdims=True) > 0.1
        b2 = b2_ref[0, 0]
        contra_max = jnp.where(anym, jax.nn.sigmoid(zmax - w2sum + b2), 0.0)
        cmask = contra_max > _THRESHOLD

        nb = (A > 0).astype(f32)
        cnt = jnp.sum(nb, axis=-1, keepdims=True)
        nbH = jnp.dot(nb, H, preferred_element_type=f32)
        neigh = nbH / jnp.maximum(cnt, 1.0)

        pre = (jnp.dot(H, rW1h_ref[...], preferred_element_type=f32)
               + jnp.dot(neigh, rW1n_ref[...], preferred_element_type=f32)
               + contra_max * rw1c_ref[...]
               + rb1_ref[...])
        act = jnp.where(pre > 0, pre, jnp.exp(pre) - 1.0)
        resolved = jnp.dot(act, rW2_ref[...], preferred_element_type=f32) + rb2_ref[...]

        apply = jnp.logical_and(cmask, cnt > 0)
        Hout_ref[bb] = jnp.where(apply, resolved, H)
        mask_ref[bb] = cmask.astype(jnp.int32)


@functools.partial(jax.jit, static_argnums=())
def kernel(H, A, det_W1, det_b1, det_W2, det_b2, res_W1, res_b1, res_W2, res_b2):
    B, N, FD = H.shape
    W1a = det_W1[:FD]
    W1b = det_W1[FD:]
    b1 = det_b1.reshape(1, FD)
    w2 = det_W2.reshape(1, FD)
    b2 = det_b2.reshape(1, 1)
    rW1h = res_W1[:FD]
    rW1n = res_W1[FD:2 * FD]
    rw1c = res_W1[2 * FD:2 * FD + 1]
    rb1 = res_b1.reshape(1, FD)
    rW2 = res_W2
    rb2 = res_b2.reshape(1, FD)

    full = lambda shape: pl.BlockSpec(shape, lambda b: (0,) * len(shape))
    Hout, mask32 = pl.pallas_call(
        _body,
        grid=(B // _BB,),
        in_specs=[
            pl.BlockSpec((_BB, N, FD), lambda b: (b, 0, 0)),
            pl.BlockSpec((_BB, N, N), lambda b: (b, 0, 0)),
            full((FD, FD)), full((FD, FD)), full((1, FD)), full((1, FD)),
            full((1, 1)),
            full((FD, FD)), full((FD, FD)), full((1, FD)), full((1, FD)),
            full((FD, FD)), full((1, FD)),
        ],
        out_specs=[
            pl.BlockSpec((_BB, N, FD), lambda b: (b, 0, 0)),
            pl.BlockSpec((_BB, N, 1), lambda b: (b, 0, 0)),
        ],
        out_shape=[
            jax.ShapeDtypeStruct((B, N, FD), jnp.float32),
            jax.ShapeDtypeStruct((B, N, 1), jnp.int32),
        ],
        compiler_params=pltpu.CompilerParams(
            dimension_semantics=("arbitrary",),
        ),
        scratch_shapes=[
            pltpu.VMEM((_BB, N, FD), jnp.float32),
            pltpu.VMEM((_BB, N, FD), jnp.float32),
            pltpu.VMEM((_BB, N, FD), jnp.float32),
            pltpu.VMEM((_BB, N, N), jnp.float32),
        ],
    )(H, A, W1a, W1b, b1, w2, b2, rW1h, rW1n, rw1c, rb1, rW2, rb2)
    return (Hout, mask32.reshape(B, N).astype(jnp.bool_))


# speedup vs baseline: 1.1241x; 1.0034x over previous
import functools

import jax
import jax.numpy as jnp
from jax.experimental import pallas as pl
from jax.experimental.pallas import tpu as pltpu

_B, _N, _FD = 4, 256, 128
_THRESHOLD = 0.5
_TI = 8
_BB = 4
_NJ = 128


def _body(H_ref, A_ref, W1a_ref, W1b_ref, b1_ref, w2_ref, b2_ref,
          rW1h_ref, rW1n_ref, rw1c_ref, rb1_ref, rW2_ref, rb2_ref,
          Hout_ref, mask_ref, XiS, PiS, EiS, zS):
    f32 = jnp.float32

    w2 = w2_ref[...]
    w2sum = jnp.sum(w2)
    NXj, Pj, Ej = [], [], []
    for bb in range(_BB):
        H = H_ref[bb]
        Xi = jnp.dot(H, W1a_ref[...], preferred_element_type=f32)
        Xj = jnp.dot(H, W1b_ref[...], preferred_element_type=f32) + b1_ref[...]
        XiS[bb] = Xi
        PiS[bb] = Xi * w2
        EiS[bb] = jnp.exp(Xi) * w2
        NXj.append(-Xj)
        Pj.append(Xj * w2 + w2)
        Ej.append(jnp.exp(Xj))

    NXj = [a[None, :, :] for a in NXj]
    Pj = [a[None, :, :] for a in Pj]
    Ej = [a[None, :, :] for a in Ej]

    def tile_fn(t, _):
        sl = pl.ds(t * _TI, _TI)
        for bb in range(_BB):
            xi = XiS[bb, sl, :][:, None, :]
            pos = PiS[bb, sl, :][:, None, :] + Pj[bb]
            neg = EiS[bb, sl, :][:, None, :] * Ej[bb]
            e = jnp.where(xi > NXj[bb], pos, neg)
            zS[bb, sl, :] = jnp.sum(e, axis=-1)
        return 0

    jax.lax.fori_loop(0, _N // _TI, tile_fn, 0, unroll=8)

    for bb in range(_BB):
        H = H_ref[bb]
        A = A_ref[bb]
        z = zS[bb]
        mask = A > 0.1
        zmax = jnp.max(jnp.where(mask, z, -1e30), axis=-1, keepdims=True)
        anym = jnp.max(A, axis=-1, keepdims=True) > 0.1
        b2 = b2_ref[0, 0]
        contra_max = jnp.where(anym, jax.nn.sigmoid(zmax - w2sum + b2), 0.0)
        cmask = contra_max > _THRESHOLD

        nb = (A > 0).astype(f32)
        cnt = jnp.sum(nb, axis=-1, keepdims=True)
        nbH = jnp.dot(nb, H, preferred_element_type=f32)
        neigh = nbH / jnp.maximum(cnt, 1.0)

        pre = (jnp.dot(H, rW1h_ref[...], preferred_element_type=f32)
               + jnp.dot(neigh, rW1n_ref[...], preferred_element_type=f32)
               + contra_max * rw1c_ref[...]
               + rb1_ref[...])
        act = jnp.where(pre > 0, pre, jnp.exp(pre) - 1.0)
        resolved = jnp.dot(act, rW2_ref[...], preferred_element_type=f32) + rb2_ref[...]

        apply = jnp.logical_and(cmask, cnt > 0)
        Hout_ref[bb] = jnp.where(apply, resolved, H)
        mask_ref[bb] = cmask.astype(jnp.int32)


@functools.partial(jax.jit, static_argnums=())
def kernel(H, A, det_W1, det_b1, det_W2, det_b2, res_W1, res_b1, res_W2, res_b2):
    B, N, FD = H.shape
    W1a = det_W1[:FD]
    W1b = det_W1[FD:]
    b1 = det_b1.reshape(1, FD)
    w2 = det_W2.reshape(1, FD)
    b2 = det_b2.reshape(1, 1)
    rW1h = res_W1[:FD]
    rW1n = res_W1[FD:2 * FD]
    rw1c = res_W1[2 * FD:2 * FD + 1]
    rb1 = res_b1.reshape(1, FD)
    rW2 = res_W2
    rb2 = res_b2.reshape(1, FD)

    full = lambda shape: pl.BlockSpec(shape, lambda b: (0,) * len(shape))
    Hout, mask32 = pl.pallas_call(
        _body,
        grid=(B // _BB,),
        in_specs=[
            pl.BlockSpec((_BB, N, FD), lambda b: (b, 0, 0)),
            pl.BlockSpec((_BB, N, N), lambda b: (b, 0, 0)),
            full((FD, FD)), full((FD, FD)), full((1, FD)), full((1, FD)),
            full((1, 1)),
            full((FD, FD)), full((FD, FD)), full((1, FD)), full((1, FD)),
            full((FD, FD)), full((1, FD)),
        ],
        out_specs=[
            pl.BlockSpec((_BB, N, FD), lambda b: (b, 0, 0)),
            pl.BlockSpec((_BB, N, 1), lambda b: (b, 0, 0)),
        ],
        out_shape=[
            jax.ShapeDtypeStruct((B, N, FD), jnp.float32),
            jax.ShapeDtypeStruct((B, N, 1), jnp.int32),
        ],
        compiler_params=pltpu.CompilerParams(
            dimension_semantics=("arbitrary",),
        ),
        scratch_shapes=[
            pltpu.VMEM((_BB, N, FD), jnp.float32),
            pltpu.VMEM((_BB, N, FD), jnp.float32),
            pltpu.VMEM((_BB, N, FD), jnp.float32),
            pltpu.VMEM((_BB, N, N), jnp.float32),
        ],
    )(H, A, W1a, W1b, b1, w2, b2, rW1h, rW1n, rw1c, rb1, rW2, rb2)
    return (Hout, mask32.reshape(B, N).astype(jnp.bool_))


# speedup vs baseline: 1.1303x; 1.0055x over previous
import functools

import jax
import jax.numpy as jnp
from jax.experimental import pallas as pl
from jax.experimental.pallas import tpu as pltpu

_B, _N, _FD = 4, 256, 128
_THRESHOLD = 0.5
_TI = 8
_BB = 4
_NJ = 128


def _body(H_ref, A_ref, W1a_ref, W1b_ref, b1_ref, w2_ref, b2_ref,
          rW1h_ref, rW1n_ref, rw1c_ref, rb1_ref, rW2_ref, rb2_ref,
          Hout_ref, mask_ref, XiS, PiS, EiS, zS):
    f32 = jnp.float32

    w2 = w2_ref[...]
    w2sum = jnp.sum(w2)
    NXj, Pj, Ej = [], [], []
    for bb in range(_BB):
        H = H_ref[bb]
        Xi = jnp.dot(H, W1a_ref[...], preferred_element_type=f32)
        Xj = jnp.dot(H, W1b_ref[...], preferred_element_type=f32) + b1_ref[...]
        XiS[bb] = Xi
        PiS[bb] = Xi * w2
        EiS[bb] = jnp.exp(Xi) * w2
        NXj.append(-Xj)
        Pj.append(Xj * w2 + w2)
        Ej.append(jnp.exp(Xj))

    NXj = [a[None, :, :] for a in NXj]
    Pj = [a[None, :, :] for a in Pj]
    Ej = [a[None, :, :] for a in Ej]

    def tile_fn(t, _):
        sl = pl.ds(t * _TI, _TI)
        for bb in range(_BB):
            xi = XiS[bb, sl, :][:, None, :]
            pos = PiS[bb, sl, :][:, None, :] + Pj[bb]
            neg = EiS[bb, sl, :][:, None, :] * Ej[bb]
            e = jnp.where(xi > NXj[bb], pos, neg)
            zS[bb, sl, :] = jnp.sum(e, axis=-1)
        return 0

    jax.lax.fori_loop(0, _N // _TI, tile_fn, 0, unroll=16)

    for bb in range(_BB):
        H = H_ref[bb]
        A = A_ref[bb]
        z = zS[bb]
        mask = A > 0.1
        zmax = jnp.max(jnp.where(mask, z, -1e30), axis=-1, keepdims=True)
        anym = jnp.max(A, axis=-1, keepdims=True) > 0.1
        b2 = b2_ref[0, 0]
        contra_max = jnp.where(anym, jax.nn.sigmoid(zmax - w2sum + b2), 0.0)
        cmask = contra_max > _THRESHOLD

        nb = (A > 0).astype(f32)
        cnt = jnp.sum(nb, axis=-1, keepdims=True)
        nbH = jnp.dot(nb, H, preferred_element_type=f32)
        neigh = nbH / jnp.maximum(cnt, 1.0)

        pre = (jnp.dot(H, rW1h_ref[...], preferred_element_type=f32)
               + jnp.dot(neigh, rW1n_ref[...], preferred_element_type=f32)
               + contra_max * rw1c_ref[...]
               + rb1_ref[...])
        act = jnp.where(pre > 0, pre, jnp.exp(pre) - 1.0)
        resolved = jnp.dot(act, rW2_ref[...], preferred_element_type=f32) + rb2_ref[...]

        apply = jnp.logical_and(cmask, cnt > 0)
        Hout_ref[bb] = jnp.where(apply, resolved, H)
        mask_ref[bb] = cmask.astype(jnp.int32)


@functools.partial(jax.jit, static_argnums=())
def kernel(H, A, det_W1, det_b1, det_W2, det_b2, res_W1, res_b1, res_W2, res_b2):
    B, N, FD = H.shape
    W1a = det_W1[:FD]
    W1b = det_W1[FD:]
    b1 = det_b1.reshape(1, FD)
    w2 = det_W2.reshape(1, FD)
    b2 = det_b2.reshape(1, 1)
    rW1h = res_W1[:FD]
    rW1n = res_W1[FD:2 * FD]
    rw1c = res_W1[2 * FD:2 * FD + 1]
    rb1 = res_b1.reshape(1, FD)
    rW2 = res_W2
    rb2 = res_b2.reshape(1, FD)

    full = lambda shape: pl.BlockSpec(shape, lambda b: (0,) * len(shape))
    Hout, mask32 = pl.pallas_call(
        _body,
        grid=(B // _BB,),
        in_specs=[
            pl.BlockSpec((_BB, N, FD), lambda b: (b, 0, 0)),
            pl.BlockSpec((_BB, N, N), lambda b: (b, 0, 0)),
            full((FD, FD)), full((FD, FD)), full((1, FD)), full((1, FD)),
            full((1, 1)),
            full((FD, FD)), full((FD, FD)), full((1, FD)), full((1, FD)),
            full((FD, FD)), full((1, FD)),
        ],
        out_specs=[
            pl.BlockSpec((_BB, N, FD), lambda b: (b, 0, 0)),
            pl.BlockSpec((_BB, N, 1), lambda b: (b, 0, 0)),
        ],
        out_shape=[
            jax.ShapeDtypeStruct((B, N, FD), jnp.float32),
            jax.ShapeDtypeStruct((B, N, 1), jnp.int32),
        ],
        compiler_params=pltpu.CompilerParams(
            dimension_semantics=("arbitrary",),
        ),
        scratch_shapes=[
            pltpu.VMEM((_BB, N, FD), jnp.float32),
            pltpu.VMEM((_BB, N, FD), jnp.float32),
            pltpu.VMEM((_BB, N, FD), jnp.float32),
            pltpu.VMEM((_BB, N, N), jnp.float32),
        ],
    )(H, A, W1a, W1b, b1, w2, b2, rW1h, rW1n, rw1c, rb1, rW2, rb2)
    return (Hout, mask32.reshape(B, N).astype(jnp.bool_))
